# baseline (speedup 1.0000x reference)
import functools

import jax
import jax.numpy as jnp
from jax import lax
from jax.experimental import pallas as pl
from jax.experimental.pallas import tpu as pltpu

_NEG = -1.0e30


def _stats_body(pooled_ref, w_ref, b_ref, s_ref, m_ref, l_ref, *, nvt):
    j = pl.program_id(1)
    logits = lax.dot_general(pooled_ref[...], w_ref[0],
                             (((1,), (1,)), ((), ())),
                             preferred_element_type=jnp.float32) + b_ref[0]

    @pl.when(j == 0)
    def _init():
        m_ref[...] = jnp.full_like(m_ref, _NEG)
        l_ref[...] = jnp.zeros_like(l_ref)

    m_old = m_ref[...]
    m_new = jnp.maximum(m_old, jnp.max(logits, axis=1, keepdims=True))
    l_ref[...] = (l_ref[...] * jnp.exp(m_old - m_new)
                  + jnp.sum(jnp.exp(logits - m_new), axis=1, keepdims=True))
    m_ref[...] = m_new

    @pl.when(j == nvt - 1)
    def _finish():
        s_ref[...] = m_ref[...] + jnp.log(l_ref[...])


def _write_body(pooled_ref, w_ref, b_ref, s_ref, out_ref, stage_ref, sems,
                *, nbands, nj, r, vt, vtail, depth):
    band = pl.program_id(0)
    j = pl.program_id(1)
    slot = lax.rem(band, depth)

    @pl.when((j == 0) & (band >= depth))
    def _reclaim():
        pltpu.make_async_copy(
            stage_ref.at[slot],
            out_ref.at[pl.ds(0, r), :],
            sems.at[slot],
        ).wait()

    y = lax.dot_general(pooled_ref[...], w_ref[0],
                        (((1,), (1,)), ((), ())),
                        preferred_element_type=jnp.float32) + b_ref[0]
    y = y - s_ref[...]

    @pl.when(j < nj - 1)
    def _store_full():
        stage_ref[slot, :, pl.ds(j * vt, vt)] = y

    @pl.when(j == nj - 1)
    def _store_tail_and_send():
        stage_ref[slot, :, pl.ds(j * vt, vtail)] = y[:, :vtail]
        pltpu.make_async_copy(
            stage_ref.at[slot],
            out_ref.at[pl.ds(band * r, r), :],
            sems.at[slot],
        ).start()

    @pl.when((band == nbands - 1) & (j == nj - 1))
    def _drain():
        for d in range(depth):
            pltpu.make_async_copy(
                stage_ref.at[d],
                out_ref.at[pl.ds(0, r), :],
                sems.at[d],
            ).wait()


def _fused_proj_logsoftmax(pooled, W, b, *, bt_s=1024, vt=2048,
                           r=64, depth=2):
    B, E = pooled.shape
    V = W.shape[0]
    pooled = pooled.astype(jnp.bfloat16)

    nvt = -(-V // vt)
    v_pad = nvt * vt
    vtail = V - (nvt - 1) * vt
    w_r = jnp.pad(W, ((0, v_pad - V), (0, 0))).astype(
        jnp.bfloat16).reshape(nvt, vt, E)
    b_r = jnp.pad(b, (0, v_pad - V),
                  constant_values=_NEG).reshape(nvt, 1, vt)

    assert B % bt_s == 0
    nb_s = B // bt_s

    s = pl.pallas_call(
        functools.partial(_stats_body, nvt=nvt),
        grid=(nb_s, nvt),
        in_specs=[
            pl.BlockSpec((bt_s, E), lambda i, j: (i, 0)),
            pl.BlockSpec((1, vt, E), lambda i, j: (j, 0, 0)),
            pl.BlockSpec((1, 1, vt), lambda i, j: (j, 0, 0)),
        ],
        out_specs=pl.BlockSpec((bt_s, 1), lambda i, j: (i, 0)),
        out_shape=jax.ShapeDtypeStruct((B, 1), jnp.float32),
        scratch_shapes=[
            pltpu.VMEM((bt_s, 1), jnp.float32),
            pltpu.VMEM((bt_s, 1), jnp.float32),
        ],
        compiler_params=pltpu.CompilerParams(
            dimension_semantics=("arbitrary", "arbitrary"),
        ),
    )(pooled, w_r, b_r)

    assert B % r == 0
    nbands = B // r
    assert nbands >= depth

    return pl.pallas_call(
        functools.partial(_write_body, nbands=nbands, nj=nvt, r=r, vt=vt,
                          vtail=vtail, depth=depth),
        grid=(nbands, nvt),
        in_specs=[
            pl.BlockSpec((r, E), lambda i, j: (i, 0)),
            pl.BlockSpec((1, vt, E), lambda i, j: (j, 0, 0)),
            pl.BlockSpec((1, 1, vt), lambda i, j: (j, 0, 0)),
            pl.BlockSpec((r, 1), lambda i, j: (i, 0)),
        ],
        out_specs=pl.BlockSpec(memory_space=pl.ANY),
        out_shape=jax.ShapeDtypeStruct((B, V), jnp.float32),
        scratch_shapes=[
            pltpu.VMEM((depth, r, V), jnp.float32),
            pltpu.SemaphoreType.DMA((depth,)),
        ],
        compiler_params=pltpu.CompilerParams(
            dimension_semantics=("arbitrary", "arbitrary"),
        ),
    )(pooled, w_r, b_r, s)


def kernel(inputs, table, W, b):
    pooled = jnp.mean(jnp.take(table, inputs, axis=0), axis=1)
    return _fused_proj_logsoftmax(pooled, W, b)


# speedup vs baseline: 1.7388x; 1.7388x over previous
import functools

import jax
import jax.numpy as jnp
from jax import lax
from jax.experimental import pallas as pl
from jax.experimental.pallas import tpu as pltpu

_NEG = -1.0e30


def _stats_body(pooled_ref, w_ref, b_ref, s_ref, m_ref, l_ref, *, nvt):
    j = pl.program_id(1)
    logits = lax.dot_general(pooled_ref[...], w_ref[0],
                             (((1,), (0,)), ((), ())),
                             preferred_element_type=jnp.float32) + b_ref[0]

    @pl.when(j == 0)
    def _init():
        m_ref[...] = jnp.full_like(m_ref, _NEG)
        l_ref[...] = jnp.zeros_like(l_ref)

    m_old = m_ref[...]
    m_new = jnp.maximum(m_old, jnp.max(logits, axis=1, keepdims=True))
    l_ref[...] = (l_ref[...] * jnp.exp(m_old - m_new)
                  + jnp.sum(jnp.exp(logits - m_new), axis=1, keepdims=True))
    m_ref[...] = m_new

    @pl.when(j == nvt - 1)
    def _finish():
        s_ref[...] = m_ref[...] + jnp.log(l_ref[...])


def _write_body(pooled_ref, w_hbm, b_hbm, s_ref, out_ref, stage_ref, sems,
                w_ref, b_ref, wsem, *, nbands, nvt, r, vt, V, depth):
    band = pl.program_id(0)
    slot = lax.rem(band, depth)

    @pl.when(band == 0)
    def _load_weights():
        pltpu.make_async_copy(w_hbm, w_ref, wsem).start()
        pltpu.make_async_copy(b_hbm, b_ref, wsem).start()
        pltpu.make_async_copy(w_hbm, w_ref, wsem).wait()
        pltpu.make_async_copy(b_hbm, b_ref, wsem).wait()

    @pl.when(band >= depth)
    def _reclaim():
        pltpu.make_async_copy(
            stage_ref.at[slot],
            out_ref.at[pl.ds(0, r), :],
            sems.at[slot],
        ).wait()

    x = pooled_ref[...]
    s = s_ref[...]
    for j in range(nvt):
        y = lax.dot_general(x, w_ref[j], (((1,), (0,)), ((), ())),
                            preferred_element_type=jnp.float32)
        y = y + b_ref[j] - s
        width = min(vt, V - j * vt)
        stage_ref[slot, :, pl.ds(j * vt, width)] = y[:, :width]

    pltpu.make_async_copy(
        stage_ref.at[slot],
        out_ref.at[pl.ds(band * r, r), :],
        sems.at[slot],
    ).start()

    @pl.when(band == nbands - 1)
    def _drain():
        for d in range(depth):
            pltpu.make_async_copy(
                stage_ref.at[d],
                out_ref.at[pl.ds(0, r), :],
                sems.at[d],
            ).wait()


def _fused_proj_logsoftmax(pooled, W, b, *, bt_s=256, vt=8192,
                           r=64, depth=2):
    B, E = pooled.shape
    V = W.shape[0]
    pooled = pooled.astype(jnp.bfloat16)

    nvt = -(-V // vt)
    v_pad = nvt * vt
    w_r = jnp.pad(W, ((0, v_pad - V), (0, 0))).astype(
        jnp.bfloat16).reshape(nvt, vt, E).transpose(0, 2, 1)
    b_r = jnp.pad(b, (0, v_pad - V),
                  constant_values=_NEG).reshape(nvt, 1, vt)

    assert B % bt_s == 0
    nb_s = B // bt_s

    s = pl.pallas_call(
        functools.partial(_stats_body, nvt=nvt),
        grid=(nb_s, nvt),
        in_specs=[
            pl.BlockSpec((bt_s, E), lambda i, j: (i, 0)),
            pl.BlockSpec((1, E, vt), lambda i, j: (j, 0, 0)),
            pl.BlockSpec((1, 1, vt), lambda i, j: (j, 0, 0)),
        ],
        out_specs=pl.BlockSpec((bt_s, 1), lambda i, j: (i, 0)),
        out_shape=jax.ShapeDtypeStruct((B, 1), jnp.float32),
        scratch_shapes=[
            pltpu.VMEM((bt_s, 1), jnp.float32),
            pltpu.VMEM((bt_s, 1), jnp.float32),
        ],
        compiler_params=pltpu.CompilerParams(
            dimension_semantics=("arbitrary", "arbitrary"),
        ),
    )(pooled, w_r, b_r)

    assert B % r == 0
    nbands = B // r
    assert nbands >= depth

    return pl.pallas_call(
        functools.partial(_write_body, nbands=nbands, nvt=nvt, r=r, vt=vt,
                          V=V, depth=depth),
        grid=(nbands,),
        in_specs=[
            pl.BlockSpec((r, E), lambda i: (i, 0)),
            pl.BlockSpec(memory_space=pl.ANY),
            pl.BlockSpec(memory_space=pl.ANY),
            pl.BlockSpec((r, 1), lambda i: (i, 0)),
        ],
        out_specs=pl.BlockSpec(memory_space=pl.ANY),
        out_shape=jax.ShapeDtypeStruct((B, V), jnp.float32),
        scratch_shapes=[
            pltpu.VMEM((depth, r, V), jnp.float32),
            pltpu.SemaphoreType.DMA((depth,)),
            pltpu.VMEM((nvt, E, vt), jnp.bfloat16),
            pltpu.VMEM((nvt, 1, vt), jnp.float32),
            pltpu.SemaphoreType.DMA,
        ],
        compiler_params=pltpu.CompilerParams(
            dimension_semantics=("arbitrary",),
        ),
    )(pooled, w_r, b_r, s)


def kernel(inputs, table, W, b):
    pooled = jnp.mean(jnp.take(table, inputs, axis=0), axis=1)
    return _fused_proj_logsoftmax(pooled, W, b)
